# baseline (speedup 1.0000x reference)
import functools

import jax
import jax.numpy as jnp
from jax import lax
from jax.experimental import pallas as pl
from jax.experimental.pallas import tpu as pltpu
from jax.experimental.pallas import tpu_sc as plsc

_N_USERS = 1000000
_N_ITEMS = 100000
_N_OCC = 32
_MAX_TS = 128
_D = 16
_B = 4096
_L = 50

_NW = 32
_CH = 128
_ITEM_CHUNKS = (_B * _L) // (_NW * _CH)
_EX_CHUNKS = (_B * 3) // (_NW * _CH)
_HALF = _ITEM_CHUNKS // 2
_IPW = _ITEM_CHUNKS * _CH
_EPW = _EX_CHUNKS * _CH

_BB = 64
_M = _BB * _L
_R = _M // 8


_NTILE = (_N_USERS + _N_ITEMS + _N_OCC + _MAX_TS) // 128
_JBASE = _NTILE // _NW
_JREM = _NTILE % _NW


def _sc_transpose_body(featT_hbm, tpack_hbm, in_v, pack_v, sem, sem_w):
    w = lax.axis_index("s") * 2 + lax.axis_index("c")
    nj = _JBASE + 1
    base = w * nj
    last = _NTILE - 1

    ar = jax.lax.iota(jnp.int32, 16)
    idx_t = ar // 8
    idx_s = ar % 8

    def fetch(j, buf):
        pltpu.async_copy(featT_hbm.at[pl.ds(0, 8), pl.ds(128 * j, 128)],
                         in_v.at[buf, 0], sem)
        pltpu.async_copy(featT_hbm.at[pl.ds(8, 8), pl.ds(128 * j, 128)],
                         in_v.at[buf, 1], sem)

    def wait(buf):
        pltpu.make_async_copy(featT_hbm.at[pl.ds(0, 8), pl.ds(0, 128)],
                              in_v.at[buf, 0], sem).wait()
        pltpu.make_async_copy(featT_hbm.at[pl.ds(0, 8), pl.ds(0, 128)],
                              in_v.at[buf, 1], sem).wait()

    fetch(jnp.minimum(base, last), 0)

    def step(i, wait_write):
        j = jnp.minimum(base + i, last)
        buf = lax.rem(i, 2)
        fetch(jnp.minimum(base + i + 1, last), 1 - buf)
        wait(buf)
        if wait_write:
            pltpu.make_async_copy(pack_v.at[buf], tpack_hbm.at[pl.ds(0, 16)],
                                  sem_w).wait()

        @plsc.parallel_loop(0, 16, unroll=8)
        def arow(a):
            for cc in range(8):
                idx_l = jnp.broadcast_to(8 * a + cc, (16,)).astype(jnp.int32)
                vec = plsc.load_gather(in_v.at[buf], [idx_t, idx_s, idx_l])
                pack_v[buf, a, pl.ds(16 * cc, 16)] = vec
        pltpu.async_copy(pack_v.at[buf], tpack_hbm.at[pl.ds(16 * j, 16)],
                         sem_w)
        return 0

    step(0, False)
    step(1, False)
    lax.fori_loop(2, nj, lambda i, c: step(i, True), 0)
    wait(1)
    for _ in range(2):
        pltpu.make_async_copy(pack_v.at[0], tpack_hbm.at[pl.ds(0, 16)],
                              sem_w).wait()


def _sc_transpose(featT):
    f32 = jnp.float32
    run = pl.kernel(
        _sc_transpose_body,
        mesh=plsc.VectorSubcoreMesh(core_axis_name="c", subcore_axis_name="s"),
        compiler_params=pltpu.CompilerParams(use_tc_tiling_on_sc=True,
                                             needs_layout_passes=False),
        out_type=jax.ShapeDtypeStruct((_NTILE * 16, 128), f32),
        scratch_types=[
            pltpu.VMEM((2, 2, 8, 128), f32),
            pltpu.VMEM((2, 16, 128), f32),
            pltpu.SemaphoreType.DMA,
            pltpu.SemaphoreType.DMA,
        ],
    )
    return run(featT)


def _repack16(src_v, pack_v, n128):
    def body(r, c):
        for j in range(8):
            pack_v[r, pl.ds(16 * j, 16)] = src_v[8 * r + j, :]
        return c
    lax.fori_loop(0, n128, body, 0)


def _sc_gather_body(feat_hbm, bias_hbm, idxi_hbm, idxe_hbm,
                    xi_hbm, xe_hbm, bi_hbm, be_hbm,
                    idx_v, idxe_v, rows_v, pack_v, exrows_v, biv_v, exb_v,
                    sem_g, sem_b, sem_e, sem_eb):
    w = lax.axis_index("s") * 2 + lax.axis_index("c")
    bias_1d = bias_hbm.at[0]

    pltpu.sync_copy(idxi_hbm.at[w], idx_v)
    pltpu.sync_copy(idxe_hbm.at[w], idxe_v)

    for j in range(_EX_CHUNKS):
        pltpu.async_copy(feat_hbm.at[idxe_v.at[j]],
                         exrows_v.at[pl.ds(j * _CH, _CH)], sem_e)
        pltpu.async_copy(bias_1d.at[idxe_v.at[j]], exb_v.at[j], sem_eb)

    def fire_bias(j, c):
        pltpu.async_copy(bias_1d.at[idx_v.at[j]], biv_v.at[j], sem_b)
        return c
    lax.fori_loop(0, _ITEM_CHUNKS, fire_bias, 0)

    for half in range(2):
        def fire_feat(j, c, half=half):
            pltpu.async_copy(feat_hbm.at[idx_v.at[_HALF * half + j]],
                             rows_v.at[pl.ds(j * _CH, _CH)], sem_g)
            return c
        lax.fori_loop(0, _HALF, fire_feat, 0)

        def drain_feat(j, c, half=half):
            pltpu.make_async_copy(feat_hbm.at[idx_v.at[_HALF * half + j]],
                                  rows_v.at[pl.ds(j * _CH, _CH)],
                                  sem_g).wait()
            return c
        lax.fori_loop(0, _HALF, drain_feat, 0)
        _repack16(rows_v, pack_v, _HALF * 16)
        pltpu.sync_copy(
            pack_v,
            xi_hbm.at[pl.ds((2 * w + half) * _HALF * 16, _HALF * 16)])

    for j in range(_EX_CHUNKS):
        pltpu.make_async_copy(feat_hbm.at[idxe_v.at[j]],
                              exrows_v.at[pl.ds(j * _CH, _CH)], sem_e).wait()
    _repack16(exrows_v, pack_v, _EPW // 8)
    pltpu.sync_copy(pack_v.at[pl.ds(0, _EPW // 8)],
                    xe_hbm.at[pl.ds(w * (_EPW // 8), _EPW // 8)])

    for j in range(_EX_CHUNKS):
        pltpu.make_async_copy(bias_1d.at[idxe_v.at[j]], exb_v.at[j],
                              sem_eb).wait()
    pltpu.sync_copy(exb_v, be_hbm.at[pl.ds(w * _EX_CHUNKS, _EX_CHUNKS)])

    def drain_bias(j, c):
        pltpu.make_async_copy(bias_1d.at[idx_v.at[j]], biv_v.at[j],
                              sem_b).wait()
        return c
    lax.fori_loop(0, _ITEM_CHUNKS, drain_bias, 0)
    pltpu.sync_copy(biv_v,
                    bi_hbm.at[pl.ds(w * _ITEM_CHUNKS, _ITEM_CHUNKS)])


def _sc_gather(feat_emb, bias_flat, idxi, idxe):
    f32, i32 = jnp.float32, jnp.int32
    run = pl.kernel(
        _sc_gather_body,
        mesh=plsc.VectorSubcoreMesh(core_axis_name="c", subcore_axis_name="s"),
        compiler_params=pltpu.CompilerParams(use_tc_tiling_on_sc=False),
        out_type=[
            jax.ShapeDtypeStruct((_B * _L // 8, 128), f32),
            jax.ShapeDtypeStruct((_B * 3 // 8, 128), f32),
            jax.ShapeDtypeStruct((_NW * _ITEM_CHUNKS, _CH), f32),
            jax.ShapeDtypeStruct((_NW * _EX_CHUNKS, _CH), f32),
        ],
        scratch_types=[
            pltpu.VMEM((_ITEM_CHUNKS, _CH), i32),
            pltpu.VMEM((_EX_CHUNKS, _CH), i32),
            pltpu.VMEM((_HALF * _CH, _D), f32),
            pltpu.VMEM((_HALF * 16, _CH), f32),
            pltpu.VMEM((_EPW, _D), f32),
            pltpu.VMEM((_ITEM_CHUNKS, _CH), f32),
            pltpu.VMEM((_EX_CHUNKS, _CH), f32),
            pltpu.SemaphoreType.DMA,
            pltpu.SemaphoreType.DMA,
            pltpu.SemaphoreType.DMA,
            pltpu.SemaphoreType.DMA,
        ],
    )
    return run(feat_emb, bias_flat, idxi, idxe)


def _tc_body(x_ref, ex_ref,
             w1bT_ref, b1_ref, w1bd_ref, w2bd_ref, b2_ref, w3bd_ref, b3_ref,
             w4t_ref, s_ref, b4_ref, out_ref):
    f32 = jnp.float32
    ex = ex_ref[...]
    q = jnp.dot(ex, w1bT_ref[...], preferred_element_type=f32) + b1_ref[...]
    ex_sum = ex[:, :16] + ex[:, 16:32] + ex[:, 32:48]
    c = 0.5 * (jnp.sum(ex_sum * ex_sum, axis=1, keepdims=True)
               - jnp.sum(ex * ex, axis=1, keepdims=True))
    qq = jnp.concatenate([q, ex_sum, c], axis=1)

    r0 = lax.broadcasted_iota(jnp.int32, (_R, _BB), 0)
    c0 = lax.broadcasted_iota(jnp.int32, (_R, _BB), 1)
    reps = []
    for j in range(8):
        e_sel = ((8 * r0 + j) // _L == c0).astype(f32)
        reps.append(jnp.dot(e_sel, qq, preferred_element_type=f32))
    rep_w = jnp.concatenate([r[:, :64] for r in reps], axis=1)
    rep_s = jnp.concatenate([r[:, 64:80] for r in reps], axis=1)
    rep_c = jnp.concatenate([r[:, 80:81] for r in reps], axis=1)

    x = x_ref[...]
    h = jnp.dot(x, w1bd_ref[...], preferred_element_type=f32) + rep_w
    h = jnp.maximum(h, 0.0)
    h = jnp.dot(h, w2bd_ref[...], preferred_element_type=f32) + b2_ref[...]
    h = jnp.maximum(h, 0.0)
    h = jnp.dot(h, w3bd_ref[...], preferred_element_type=f32) + b3_ref[...]
    h = jnp.maximum(h, 0.0)
    v = h * w4t_ref[...] + x * rep_s
    s8 = jnp.dot(v, s_ref[...], preferred_element_type=f32) + rep_c
    s8 = s8 + b4_ref[...]
    out_ref[...] = s8


def _tc_call(xi_pack, xe_flat, w1bT, b1r, w1bd, w2bd, b2bd, w3bd, b3bd,
             w4t, s_mat, b4r, interpret=False):
    f32 = jnp.float32
    full = lambda a: pl.BlockSpec(a.shape, lambda i: (0, 0))
    return pl.pallas_call(
        _tc_body,
        grid=(_B // _BB,),
        in_specs=[
            pl.BlockSpec((_R, 128), lambda i: (i, 0)),
            pl.BlockSpec((_BB, 3 * _D), lambda i: (i, 0)),
            full(w1bT), full(b1r), full(w1bd), full(w2bd), full(b2bd),
            full(w3bd), full(b3bd), full(w4t), full(s_mat), full(b4r),
        ],
        out_specs=pl.BlockSpec((_R, 8), lambda i: (i, 0)),
        out_shape=jax.ShapeDtypeStruct((_B * _L // 8, 8), f32),
        interpret=interpret,
    )(xi_pack, xe_flat, w1bT, b1r, w1bd, w2bd, b2bd, w3bd, b3bd,
      w4t, s_mat, b4r)


def _fold_bn(W, b, g, be, eps=1e-5):
    s = g / jnp.sqrt(1.0 + eps)
    return W * s[:, None], b * s + be


def kernel(user_code, item_code, user_occupation, item_timestamp_rank,
           feat_emb, bias_emb,
           W1, b1, g1, be1, W2, b2, g2, be2, W3, b3, g3, be3, W4, b4):
    f32 = jnp.float32

    item_idx = (item_code + _N_USERS).reshape(_NW, _ITEM_CHUNKS, _CH)
    occ_idx = user_occupation + (_N_USERS + _N_ITEMS)
    ts_idx = item_timestamp_rank + (_N_USERS + _N_ITEMS + _N_OCC)
    ex_idx = jnp.stack([user_code, occ_idx, ts_idx], axis=1)
    ex_idx = ex_idx.reshape(_NW, _EX_CHUNKS, _CH)

    tpack = _sc_transpose(feat_emb.T)
    feat_lin = tpack.reshape(_N_USERS + _N_ITEMS + _N_OCC + _MAX_TS, _D)

    xi_pack, xe, bi, be3v = _sc_gather(feat_lin, bias_emb.T, item_idx, ex_idx)
    xe_flat = xe.reshape(_B, 3 * _D)

    W1f, b1f = _fold_bn(W1, b1, g1, be1)
    W2f, b2f = _fold_bn(W2, b2, g2, be2)
    W3f, b3f = _fold_bn(W3, b3, g3, be3)
    eye8 = jnp.eye(8, dtype=f32)
    w1bT = W1f[:, _D:].T
    b1r = b1f.reshape(1, -1)
    w1bd = jnp.kron(eye8, W1f[:, :_D].T)
    w2bd = jnp.kron(eye8, W2f.T)
    w3bd = jnp.kron(eye8, W3f.T)
    b2bd = jnp.tile(b2f, 8).reshape(1, -1)
    b3bd = jnp.tile(b3f, 8).reshape(1, -1)
    w4t = jnp.tile(W4.reshape(-1), 8).reshape(1, -1)
    s_mat = jnp.kron(eye8, jnp.ones((_D, 1), f32))
    b4r = b4.reshape(1, 1)

    deep_fm = _tc_call(xi_pack, xe_flat, w1bT, b1r, w1bd, w2bd, b2bd,
                       w3bd, b3bd, w4t, s_mat, b4r)

    bias_i = bi.reshape(_B, _L)
    bias_e = be3v.reshape(_B, 3).sum(axis=1, keepdims=True)
    return deep_fm.reshape(_B, _L) + bias_i + bias_e


# speedup vs baseline: 1.0074x; 1.0074x over previous
import functools

import jax
import jax.numpy as jnp
from jax import lax
from jax.experimental import pallas as pl
from jax.experimental.pallas import tpu as pltpu
from jax.experimental.pallas import tpu_sc as plsc

_N_USERS = 1000000
_N_ITEMS = 100000
_N_OCC = 32
_MAX_TS = 128
_D = 16
_B = 4096
_L = 50

_NW = 32
_CH = 128
_ITEM_CHUNKS = (_B * _L) // (_NW * _CH)
_EX_CHUNKS = (_B * 3) // (_NW * _CH)
_HALF = _ITEM_CHUNKS // 2
_IPW = _ITEM_CHUNKS * _CH
_EPW = _EX_CHUNKS * _CH

_BB = 64
_M = _BB * _L
_R = _M // 8


_NTILE = (_N_USERS + _N_ITEMS + _N_OCC + _MAX_TS) // 128
_JBASE = _NTILE // _NW
_JREM = _NTILE % _NW


def _sc_transpose_body(featT_hbm, tpack_hbm, in_v, pack_v, sem, sem_w):
    w = lax.axis_index("s") * 2 + lax.axis_index("c")
    nj = _JBASE + 1
    base = w * nj
    last = _NTILE - 1

    ar = jax.lax.iota(jnp.int32, 16)
    idx_t = ar // 8
    idx_s = ar % 8

    def fetch(j, buf):
        pltpu.async_copy(featT_hbm.at[pl.ds(0, 8), pl.ds(128 * j, 128)],
                         in_v.at[buf, 0], sem)
        pltpu.async_copy(featT_hbm.at[pl.ds(8, 8), pl.ds(128 * j, 128)],
                         in_v.at[buf, 1], sem)

    def wait(buf):
        pltpu.make_async_copy(featT_hbm.at[pl.ds(0, 8), pl.ds(0, 128)],
                              in_v.at[buf, 0], sem).wait()
        pltpu.make_async_copy(featT_hbm.at[pl.ds(0, 8), pl.ds(0, 128)],
                              in_v.at[buf, 1], sem).wait()

    fetch(jnp.minimum(base, last), 0)

    def step(i, wait_write):
        j = jnp.minimum(base + i, last)
        buf = lax.rem(i, 2)
        fetch(jnp.minimum(base + i + 1, last), 1 - buf)
        wait(buf)
        if wait_write:
            pltpu.make_async_copy(pack_v.at[buf], tpack_hbm.at[pl.ds(0, 16)],
                                  sem_w).wait()

        @plsc.parallel_loop(0, 16, unroll=4)
        def arow(a):
            for cc in range(8):
                idx_l = jnp.broadcast_to(8 * a + cc, (16,)).astype(jnp.int32)
                vec = plsc.load_gather(in_v.at[buf], [idx_t, idx_s, idx_l])
                pack_v[buf, a, pl.ds(16 * cc, 16)] = vec
        pltpu.async_copy(pack_v.at[buf], tpack_hbm.at[pl.ds(16 * j, 16)],
                         sem_w)
        return 0

    step(0, False)
    step(1, False)
    lax.fori_loop(2, nj, lambda i, c: step(i, True), 0)
    wait(1)
    for _ in range(2):
        pltpu.make_async_copy(pack_v.at[0], tpack_hbm.at[pl.ds(0, 16)],
                              sem_w).wait()


def _sc_transpose(featT):
    f32 = jnp.float32
    run = pl.kernel(
        _sc_transpose_body,
        mesh=plsc.VectorSubcoreMesh(core_axis_name="c", subcore_axis_name="s"),
        compiler_params=pltpu.CompilerParams(use_tc_tiling_on_sc=True,
                                             needs_layout_passes=False),
        out_type=jax.ShapeDtypeStruct((_NTILE * 16, 128), f32),
        scratch_types=[
            pltpu.VMEM((2, 2, 8, 128), f32),
            pltpu.VMEM((2, 16, 128), f32),
            pltpu.SemaphoreType.DMA,
            pltpu.SemaphoreType.DMA,
        ],
    )
    return run(featT)


def _repack16(src_v, pack_v, n128):
    def body(r, c):
        for j in range(8):
            pack_v[r, pl.ds(16 * j, 16)] = src_v[8 * r + j, :]
        return c
    lax.fori_loop(0, n128, body, 0)


def _sc_gather_body(feat_hbm, bias_hbm, idxi_hbm, idxe_hbm,
                    xi_hbm, xe_hbm, bi_hbm, be_hbm,
                    idx_v, idxe_v, rows_v, pack_v, exrows_v, biv_v, exb_v,
                    sem_g, sem_b, sem_e, sem_eb):
    w = lax.axis_index("s") * 2 + lax.axis_index("c")
    bias_1d = bias_hbm.at[0]

    pltpu.sync_copy(idxi_hbm.at[w], idx_v)
    pltpu.sync_copy(idxe_hbm.at[w], idxe_v)

    for j in range(_EX_CHUNKS):
        pltpu.async_copy(feat_hbm.at[idxe_v.at[j]],
                         exrows_v.at[pl.ds(j * _CH, _CH)], sem_e)
        pltpu.async_copy(bias_1d.at[idxe_v.at[j]], exb_v.at[j], sem_eb)

    def fire_bias(j, c):
        pltpu.async_copy(bias_1d.at[idx_v.at[j]], biv_v.at[j], sem_b)
        return c
    lax.fori_loop(0, _ITEM_CHUNKS, fire_bias, 0)

    for half in range(2):
        def fire_feat(j, c, half=half):
            pltpu.async_copy(feat_hbm.at[idx_v.at[_HALF * half + j]],
                             rows_v.at[pl.ds(j * _CH, _CH)], sem_g)
            return c
        lax.fori_loop(0, _HALF, fire_feat, 0)

        def drain_feat(j, c, half=half):
            pltpu.make_async_copy(feat_hbm.at[idx_v.at[_HALF * half + j]],
                                  rows_v.at[pl.ds(j * _CH, _CH)],
                                  sem_g).wait()
            return c
        lax.fori_loop(0, _HALF, drain_feat, 0)
        _repack16(rows_v, pack_v, _HALF * 16)
        pltpu.sync_copy(
            pack_v,
            xi_hbm.at[pl.ds((2 * w + half) * _HALF * 16, _HALF * 16)])

    for j in range(_EX_CHUNKS):
        pltpu.make_async_copy(feat_hbm.at[idxe_v.at[j]],
                              exrows_v.at[pl.ds(j * _CH, _CH)], sem_e).wait()
    _repack16(exrows_v, pack_v, _EPW // 8)
    pltpu.sync_copy(pack_v.at[pl.ds(0, _EPW // 8)],
                    xe_hbm.at[pl.ds(w * (_EPW // 8), _EPW // 8)])

    for j in range(_EX_CHUNKS):
        pltpu.make_async_copy(bias_1d.at[idxe_v.at[j]], exb_v.at[j],
                              sem_eb).wait()
    pltpu.sync_copy(exb_v, be_hbm.at[pl.ds(w * _EX_CHUNKS, _EX_CHUNKS)])

    def drain_bias(j, c):
        pltpu.make_async_copy(bias_1d.at[idx_v.at[j]], biv_v.at[j],
                              sem_b).wait()
        return c
    lax.fori_loop(0, _ITEM_CHUNKS, drain_bias, 0)
    pltpu.sync_copy(biv_v,
                    bi_hbm.at[pl.ds(w * _ITEM_CHUNKS, _ITEM_CHUNKS)])


def _sc_gather(feat_emb, bias_flat, idxi, idxe):
    f32, i32 = jnp.float32, jnp.int32
    run = pl.kernel(
        _sc_gather_body,
        mesh=plsc.VectorSubcoreMesh(core_axis_name="c", subcore_axis_name="s"),
        compiler_params=pltpu.CompilerParams(use_tc_tiling_on_sc=False),
        out_type=[
            jax.ShapeDtypeStruct((_B * _L // 8, 128), f32),
            jax.ShapeDtypeStruct((_B * 3 // 8, 128), f32),
            jax.ShapeDtypeStruct((_NW * _ITEM_CHUNKS, _CH), f32),
            jax.ShapeDtypeStruct((_NW * _EX_CHUNKS, _CH), f32),
        ],
        scratch_types=[
            pltpu.VMEM((_ITEM_CHUNKS, _CH), i32),
            pltpu.VMEM((_EX_CHUNKS, _CH), i32),
            pltpu.VMEM((_HALF * _CH, _D), f32),
            pltpu.VMEM((_HALF * 16, _CH), f32),
            pltpu.VMEM((_EPW, _D), f32),
            pltpu.VMEM((_ITEM_CHUNKS, _CH), f32),
            pltpu.VMEM((_EX_CHUNKS, _CH), f32),
            pltpu.SemaphoreType.DMA,
            pltpu.SemaphoreType.DMA,
            pltpu.SemaphoreType.DMA,
            pltpu.SemaphoreType.DMA,
        ],
    )
    return run(feat_emb, bias_flat, idxi, idxe)


def _tc_body(x_ref, ex_ref,
             w1bT_ref, b1_ref, w1bd_ref, w2bd_ref, b2_ref, w3bd_ref, b3_ref,
             w4t_ref, s_ref, b4_ref, out_ref):
    f32 = jnp.float32
    ex = ex_ref[...]
    q = jnp.dot(ex, w1bT_ref[...], preferred_element_type=f32) + b1_ref[...]
    ex_sum = ex[:, :16] + ex[:, 16:32] + ex[:, 32:48]
    c = 0.5 * (jnp.sum(ex_sum * ex_sum, axis=1, keepdims=True)
               - jnp.sum(ex * ex, axis=1, keepdims=True))
    qq = jnp.concatenate([q, ex_sum, c], axis=1)

    r0 = lax.broadcasted_iota(jnp.int32, (_R, _BB), 0)
    c0 = lax.broadcasted_iota(jnp.int32, (_R, _BB), 1)
    reps = []
    for j in range(8):
        e_sel = ((8 * r0 + j) // _L == c0).astype(f32)
        reps.append(jnp.dot(e_sel, qq, preferred_element_type=f32))
    rep_w = jnp.concatenate([r[:, :64] for r in reps], axis=1)
    rep_s = jnp.concatenate([r[:, 64:80] for r in reps], axis=1)
    rep_c = jnp.concatenate([r[:, 80:81] for r in reps], axis=1)

    x = x_ref[...]
    h = jnp.dot(x, w1bd_ref[...], preferred_element_type=f32) + rep_w
    h = jnp.maximum(h, 0.0)
    h = jnp.dot(h, w2bd_ref[...], preferred_element_type=f32) + b2_ref[...]
    h = jnp.maximum(h, 0.0)
    h = jnp.dot(h, w3bd_ref[...], preferred_element_type=f32) + b3_ref[...]
    h = jnp.maximum(h, 0.0)
    v = h * w4t_ref[...] + x * rep_s
    s8 = jnp.dot(v, s_ref[...], preferred_element_type=f32) + rep_c
    s8 = s8 + b4_ref[...]
    out_ref[...] = s8


def _tc_call(xi_pack, xe_flat, w1bT, b1r, w1bd, w2bd, b2bd, w3bd, b3bd,
             w4t, s_mat, b4r, interpret=False):
    f32 = jnp.float32
    full = lambda a: pl.BlockSpec(a.shape, lambda i: (0, 0))
    return pl.pallas_call(
        _tc_body,
        grid=(_B // _BB,),
        in_specs=[
            pl.BlockSpec((_R, 128), lambda i: (i, 0)),
            pl.BlockSpec((_BB, 3 * _D), lambda i: (i, 0)),
            full(w1bT), full(b1r), full(w1bd), full(w2bd), full(b2bd),
            full(w3bd), full(b3bd), full(w4t), full(s_mat), full(b4r),
        ],
        out_specs=pl.BlockSpec((_R, 8), lambda i: (i, 0)),
        out_shape=jax.ShapeDtypeStruct((_B * _L // 8, 8), f32),
        interpret=interpret,
    )(xi_pack, xe_flat, w1bT, b1r, w1bd, w2bd, b2bd, w3bd, b3bd,
      w4t, s_mat, b4r)


def _fold_bn(W, b, g, be, eps=1e-5):
    s = g / jnp.sqrt(1.0 + eps)
    return W * s[:, None], b * s + be


def kernel(user_code, item_code, user_occupation, item_timestamp_rank,
           feat_emb, bias_emb,
           W1, b1, g1, be1, W2, b2, g2, be2, W3, b3, g3, be3, W4, b4):
    f32 = jnp.float32

    item_idx = (item_code + _N_USERS).reshape(_NW, _ITEM_CHUNKS, _CH)
    occ_idx = user_occupation + (_N_USERS + _N_ITEMS)
    ts_idx = item_timestamp_rank + (_N_USERS + _N_ITEMS + _N_OCC)
    ex_idx = jnp.stack([user_code, occ_idx, ts_idx], axis=1)
    ex_idx = ex_idx.reshape(_NW, _EX_CHUNKS, _CH)

    tpack = _sc_transpose(feat_emb.T)
    feat_lin = tpack.reshape(_N_USERS + _N_ITEMS + _N_OCC + _MAX_TS, _D)

    xi_pack, xe, bi, be3v = _sc_gather(feat_lin, bias_emb.T, item_idx, ex_idx)
    xe_flat = xe.reshape(_B, 3 * _D)

    W1f, b1f = _fold_bn(W1, b1, g1, be1)
    W2f, b2f = _fold_bn(W2, b2, g2, be2)
    W3f, b3f = _fold_bn(W3, b3, g3, be3)
    eye8 = jnp.eye(8, dtype=f32)
    w1bT = W1f[:, _D:].T
    b1r = b1f.reshape(1, -1)
    w1bd = jnp.kron(eye8, W1f[:, :_D].T)
    w2bd = jnp.kron(eye8, W2f.T)
    w3bd = jnp.kron(eye8, W3f.T)
    b2bd = jnp.tile(b2f, 8).reshape(1, -1)
    b3bd = jnp.tile(b3f, 8).reshape(1, -1)
    w4t = jnp.tile(W4.reshape(-1), 8).reshape(1, -1)
    s_mat = jnp.kron(eye8, jnp.ones((_D, 1), f32))
    b4r = b4.reshape(1, 1)

    deep_fm = _tc_call(xi_pack, xe_flat, w1bT, b1r, w1bd, w2bd, b2bd,
                       w3bd, b3bd, w4t, s_mat, b4r)

    bias_i = bi.reshape(_B, _L)
    bias_e = be3v.reshape(_B, 3).sum(axis=1, keepdims=True)
    return deep_fm.reshape(_B, _L) + bias_i + bias_e
